# Initial kernel scaffold; baseline (speedup 1.0000x reference)
import jax
import jax.numpy as jnp
from jax.experimental import pallas as pl


def kernel(x, edge_list, edge_attr, W, b, attn, fc_w, fc_b):
    raise NotImplementedError("write your pallas kernel here")


# speedup vs baseline: 77.8795x; 77.8795x over previous
import jax
import jax.numpy as jnp
from jax import lax
from jax.experimental import pallas as pl
from jax.experimental.pallas import tpu as pltpu
from jax.experimental.pallas import tpu_sc as plsc

N = 10000
F_IN = 128
E = 320000
NC = 2
NS = 16
L = 16
NP = 10240
CHUNK = NP // NS
EPAD = 327680
EROWS = EPAD // 128
WROWS = EROWS // (NC * NS)
NSORT = 16384
SROWS = NSORT // 128
OROWS = NP // 128
INT_MIN = -(2**31)


def _mesh():
    return plsc.VectorSubcoreMesh(core_axis_name="c", subcore_axis_name="s")


def _deg_body(col2, ew2, degp, colb, ewb, ta, deg_s, sem):
    c = lax.axis_index("c")
    s = lax.axis_index("s")
    w = c * NS + s

    def zl(i, _):
        ta[pl.ds(i * L, L)] = jnp.zeros((L,), jnp.float32)
        return 0
    lax.fori_loop(0, CHUNK // L, zl, 0)
    pltpu.sync_copy(ta, deg_s.at[pl.ds(s * CHUNK, CHUNK)])
    plsc.subcore_barrier()

    pltpu.sync_copy(col2.at[pl.ds(w * WROWS, WROWS)], colb)
    pltpu.sync_copy(ew2.at[pl.ds(w * WROWS, WROWS)], ewb)

    def dscat(j, _):
        pltpu.async_copy(ewb.at[j], deg_s.at[colb.at[j]], sem, add=True)
        return 0
    lax.fori_loop(0, WROWS, dscat, 0)
    pltpu.make_async_copy(ew2.at[pl.ds(0, WROWS)], ewb, sem).wait()
    plsc.subcore_barrier()

    pltpu.sync_copy(deg_s.at[pl.ds(s * CHUNK, CHUNK)], ta)
    pltpu.sync_copy(ta, degp.at[c, pl.ds(s * CHUNK, CHUNK)])


def _deg_call(col2, ew2):
    return pl.kernel(
        _deg_body,
        out_type=jax.ShapeDtypeStruct((NC, NP), jnp.float32),
        mesh=_mesh(),
        compiler_params=pltpu.CompilerParams(needs_layout_passes=False),
        scratch_types=[
            pltpu.VMEM((WROWS, 128), jnp.int32),
            pltpu.VMEM((WROWS, 128), jnp.float32),
            pltpu.VMEM((CHUNK,), jnp.float32),
            pltpu.VMEM_SHARED((NP,), jnp.float32),
            pltpu.SemaphoreType.DMA,
        ],
    )(col2, ew2)


def _mm_body(wt_ref, x_ref, degp_ref, ht_ref, dinv_ref):
    ht_ref[...] = lax.dot_general(
        wt_ref[...], x_ref[...], (((1,), (1,)), ((), ())),
        preferred_element_type=jnp.float32)
    deg = degp_ref[0] + degp_ref[1] + 1.0
    dinv_ref[...] = 1.0 / jnp.sqrt(deg)


def _matmul(wt, x_pad, degp):
    return pl.pallas_call(
        _mm_body,
        out_shape=(
            jax.ShapeDtypeStruct((2, NP), jnp.float32),
            jax.ShapeDtypeStruct((OROWS, 128), jnp.float32),
        ),
    )(wt, x_pad, degp)


def _msg_body(row2, col2, ew2, ht, dinv, out_hbm,
              rowb, colb, ewb, m0, m1, dinv_v, h0_v, h1_v, ta, tb,
              out0_s, out1_s, sem):
    c = lax.axis_index("c")
    s = lax.axis_index("s")
    w = c * NS + s

    is0 = jnp.where(c == 0, jnp.float32(1.0), jnp.float32(0.0))
    pltpu.sync_copy(dinv.at[pl.ds(s * CHUNK, CHUNK)], ta)
    pltpu.sync_copy(ht.at[0, pl.ds(s * CHUNK, CHUNK)], tb)

    def il0(i, _):
        y = ta[pl.ds(i * L, L)]
        tb[pl.ds(i * L, L)] = is0 * (y * y) * tb[pl.ds(i * L, L)]
        return 0
    lax.fori_loop(0, CHUNK // L, il0, 0)
    pltpu.sync_copy(tb, out0_s.at[pl.ds(s * CHUNK, CHUNK)])
    pltpu.sync_copy(ht.at[1, pl.ds(s * CHUNK, CHUNK)], tb)

    def il1(i, _):
        y = ta[pl.ds(i * L, L)]
        tb[pl.ds(i * L, L)] = is0 * (y * y) * tb[pl.ds(i * L, L)]
        return 0
    lax.fori_loop(0, CHUNK // L, il1, 0)
    pltpu.sync_copy(tb, out1_s.at[pl.ds(s * CHUNK, CHUNK)])
    plsc.subcore_barrier()

    pltpu.sync_copy(dinv, dinv_v)
    pltpu.sync_copy(ht.at[0], h0_v)
    pltpu.sync_copy(ht.at[1], h1_v)
    pltpu.sync_copy(row2.at[pl.ds(w * WROWS, WROWS)], rowb)
    pltpu.sync_copy(col2.at[pl.ds(w * WROWS, WROWS)], colb)
    pltpu.sync_copy(ew2.at[pl.ds(w * WROWS, WROWS)], ewb)

    def ml(i, _):
        def mlj(j, _):
            r16 = rowb[i, pl.ds(j * L, L)]
            c16 = colb[i, pl.ds(j * L, L)]
            w16 = ewb[i, pl.ds(j * L, L)]
            dr = plsc.load_gather(dinv_v, [r16])
            dc = plsc.load_gather(dinv_v, [c16])
            nv = dr * w16 * dc
            h0g = plsc.load_gather(h0_v, [r16])
            h1g = plsc.load_gather(h1_v, [r16])
            m0[i, pl.ds(j * L, L)] = nv * h0g
            m1[i, pl.ds(j * L, L)] = nv * h1g
            return 0
        lax.fori_loop(0, 128 // L, mlj, 0)
        return 0
    lax.fori_loop(0, WROWS, ml, 0)

    def mscat(j, _):
        pltpu.async_copy(m0.at[j], out0_s.at[colb.at[j]], sem, add=True)
        pltpu.async_copy(m1.at[j], out1_s.at[colb.at[j]], sem, add=True)
        return 0
    lax.fori_loop(0, WROWS, mscat, 0)
    pltpu.make_async_copy(ew2.at[pl.ds(0, WROWS)], m0, sem).wait()
    pltpu.make_async_copy(ew2.at[pl.ds(0, WROWS)], m1, sem).wait()
    plsc.subcore_barrier()

    pltpu.sync_copy(out0_s.at[pl.ds(s * CHUNK, CHUNK)], ta)
    pltpu.sync_copy(ta, out_hbm.at[c, 0, pl.ds(s * CHUNK, CHUNK)])
    pltpu.sync_copy(out1_s.at[pl.ds(s * CHUNK, CHUNK)], ta)
    pltpu.sync_copy(ta, out_hbm.at[c, 1, pl.ds(s * CHUNK, CHUNK)])


def _msg_call(row2, col2, ew2, ht, dinv1d):
    return pl.kernel(
        _msg_body,
        out_type=jax.ShapeDtypeStruct((NC, 2, NP), jnp.float32),
        mesh=_mesh(),
        compiler_params=pltpu.CompilerParams(needs_layout_passes=False),
        scratch_types=[
            pltpu.VMEM((WROWS, 128), jnp.int32),
            pltpu.VMEM((WROWS, 128), jnp.int32),
            pltpu.VMEM((WROWS, 128), jnp.float32),
            pltpu.VMEM((WROWS, 128), jnp.float32),
            pltpu.VMEM((WROWS, 128), jnp.float32),
            pltpu.VMEM((NP,), jnp.float32),
            pltpu.VMEM((NP,), jnp.float32),
            pltpu.VMEM((NP,), jnp.float32),
            pltpu.VMEM((CHUNK,), jnp.float32),
            pltpu.VMEM((CHUNK,), jnp.float32),
            pltpu.VMEM_SHARED((NP,), jnp.float32),
            pltpu.VMEM_SHARED((NP,), jnp.float32),
            pltpu.SemaphoreType.DMA,
        ],
    )(row2, col2, ew2, ht, dinv1d)


def _topk_body(op_ref, par_ref, f0_ref, f1_ref, o_ref):
    par = par_ref[...]
    b0, b1 = par[0, 0], par[0, 1]
    a0, a1 = par[0, 2], par[0, 3]
    fcb = par[0, 4]
    na = jnp.sqrt(a0 * a0 + a1 * a1)

    o0 = jnp.maximum(op_ref[0, 0] + op_ref[1, 0] + b0, 0.0)
    o1 = jnp.maximum(op_ref[0, 1] + op_ref[1, 1] + b1, 0.0)
    z = (o0 * a0 + o1 * a1) / na
    score = jnp.tanh(z)

    rr = lax.broadcasted_iota(jnp.int32, (OROWS, 128), 0)
    cc = lax.broadcasted_iota(jnp.int32, (OROWS, 128), 1)
    valid = (rr * 128 + cc) < N
    sb = lax.bitcast_convert_type(z, jnp.int32)
    key = jnp.where(sb >= 0, sb, jnp.bitwise_xor(~sb, jnp.int32(INT_MIN)))
    key = jnp.where(valid, key, jnp.int32(INT_MIN))
    p0 = jnp.where(valid, score * o0, 0.0)
    p1 = jnp.where(valid, score * o1, 0.0)

    pad_i = jnp.full((SROWS - OROWS, 128), INT_MIN, jnp.int32)
    pad_f = jnp.zeros((SROWS - OROWS, 128), jnp.float32)
    K = jnp.concatenate([key, pad_i], axis=0)
    P0 = jnp.concatenate([p0, pad_f], axis=0)
    P1 = jnp.concatenate([p1, pad_f], axis=0)
    R = lax.broadcasted_iota(jnp.int32, (SROWS, 128), 0)
    C = lax.broadcasted_iota(jnp.int32, (SROWS, 128), 1)
    I = R * 128 + C

    def xshuf(x, j):
        if j < 128:
            lo = (C & j) == 0
            return jnp.where(lo, pltpu.roll(x, 128 - j, 1),
                             pltpu.roll(x, j, 1))
        m = j // 128
        lo = (R & m) == 0
        return jnp.where(lo, pltpu.roll(x, SROWS - m, 0),
                         pltpu.roll(x, m, 0))

    def bit_set(j):
        return ((C & j) != 0) if j < 128 else ((R & (j // 128)) != 0)

    k = 2
    while k <= NSORT:
        j = k // 2
        while j >= 1:
            Kp, Ip = xshuf(K, j), xshuf(I, j)
            P0p, P1p = xshuf(P0, j), xshuf(P1, j)
            before = (K > Kp) | ((K == Kp) & (I < Ip))
            is_low = ~bit_set(j)
            dir_asc = ~bit_set(k)
            cond = before == (is_low == dir_asc)
            K = jnp.where(cond, K, Kp)
            I = jnp.where(cond, I, Ip)
            P0 = jnp.where(cond, P0, P0p)
            P1 = jnp.where(cond, P1, P1p)
            j //= 2
        k *= 2

    ypre = jnp.sum(P0 * f0_ref[...] + P1 * f1_ref[...]) + fcb
    y = jnp.float32(1.0) / (jnp.float32(1.0) + jnp.exp(-ypre))
    o_ref[...] = jnp.full((8, 128), y, jnp.float32)


def _topk(out_part, params, f0, f1):
    return pl.pallas_call(
        _topk_body,
        out_shape=jax.ShapeDtypeStruct((8, 128), jnp.float32),
    )(out_part, params, f0, f1)


def kernel(x, edge_list, edge_attr, W, b, attn, fc_w, fc_b):
    row = edge_list[0].astype(jnp.int32)
    col = edge_list[1].astype(jnp.int32)
    ew = edge_attr.astype(jnp.float32)

    npad = EPAD - E
    rowp = jnp.concatenate([row, jnp.zeros((npad,), jnp.int32)])
    colp = jnp.concatenate(
        [col, N + (jnp.arange(npad, dtype=jnp.int32) % (NP - N))])
    ewp = jnp.concatenate([ew, jnp.zeros((npad,), jnp.float32)])
    row2 = rowp.reshape(EROWS, 128)
    col2 = colp.reshape(EROWS, 128)
    ew2 = ewp.reshape(EROWS, 128)

    x_pad = jnp.pad(x, ((0, NP - N), (0, 0)))
    wt = W.T

    degp = _deg_call(col2, ew2)
    ht, dinv2d = _matmul(wt, x_pad, degp.reshape(NC, OROWS, 128))
    dinv1d = dinv2d.reshape(NP)
    out_part = _msg_call(row2, col2, ew2, ht, dinv1d)

    params = jnp.zeros((8, 128), jnp.float32)
    params = params.at[0, 0].set(b[0]).at[0, 1].set(b[1])
    params = params.at[0, 2].set(attn[0]).at[0, 3].set(attn[1])
    params = params.at[0, 4].set(fc_b[0])

    fr = fc_w.reshape(N, 2)
    f0 = jnp.pad(fr[:, 0], (0, NSORT - N)).reshape(SROWS, 128)
    f1 = jnp.pad(fr[:, 1], (0, NSORT - N)).reshape(SROWS, 128)

    yblk = _topk(out_part.reshape(NC, 2, OROWS, 128), params, f0, f1)
    return yblk[0, 0].reshape(1)


# speedup vs baseline: 93.3777x; 1.1990x over previous
import jax
import jax.numpy as jnp
from jax import lax
from jax.experimental import pallas as pl
from jax.experimental.pallas import tpu as pltpu
from jax.experimental.pallas import tpu_sc as plsc

N = 10000
F_IN = 128
E = 320000
NC = 2
NS = 16
L = 16
NP = 10240
CHUNK = NP // NS
EPAD = 327680
EROWS = EPAD // 128
WROWS = EROWS // (NC * NS)
NSORT = 16384
SROWS = NSORT // 128
OROWS = NP // 128
INT_MIN = -(2**31)


def _mesh():
    return plsc.VectorSubcoreMesh(core_axis_name="c", subcore_axis_name="s")


def _deg_body(col2, ew2, degp, colb, ewb, ta, deg_s, sem):
    c = lax.axis_index("c")
    s = lax.axis_index("s")
    w = c * NS + s

    def zl(i, _):
        ta[pl.ds(i * L, L)] = jnp.zeros((L,), jnp.float32)
        return 0
    lax.fori_loop(0, CHUNK // L, zl, 0)
    pltpu.sync_copy(ta, deg_s.at[pl.ds(s * CHUNK, CHUNK)])
    plsc.subcore_barrier()

    pltpu.sync_copy(col2.at[pl.ds(w * WROWS, WROWS)], colb)
    pltpu.sync_copy(ew2.at[pl.ds(w * WROWS, WROWS)], ewb)

    def dscat(j, _):
        pltpu.async_copy(ewb.at[j], deg_s.at[colb.at[j]], sem, add=True)
        return 0
    lax.fori_loop(0, WROWS, dscat, 0)
    pltpu.make_async_copy(ew2.at[pl.ds(0, WROWS)], ewb, sem).wait()
    plsc.subcore_barrier()

    pltpu.sync_copy(deg_s.at[pl.ds(s * CHUNK, CHUNK)], ta)
    pltpu.sync_copy(ta, degp.at[c, pl.ds(s * CHUNK, CHUNK)])


def _deg_call(col2, ew2):
    return pl.kernel(
        _deg_body,
        out_type=jax.ShapeDtypeStruct((NC, NP), jnp.float32),
        mesh=_mesh(),
        compiler_params=pltpu.CompilerParams(needs_layout_passes=False),
        scratch_types=[
            pltpu.VMEM((WROWS, 128), jnp.int32),
            pltpu.VMEM((WROWS, 128), jnp.float32),
            pltpu.VMEM((CHUNK,), jnp.float32),
            pltpu.VMEM_SHARED((NP,), jnp.float32),
            pltpu.SemaphoreType.DMA,
        ],
    )(col2, ew2)


def _mm_body(wt_ref, x_ref, degp_ref, g_ref, dinv_ref):
    ht = lax.dot_general(
        wt_ref[...], x_ref[...], (((1,), (1,)), ((), ())),
        preferred_element_type=jnp.float32)
    deg = degp_ref[0] + degp_ref[1] + 1.0
    dinv = 1.0 / jnp.sqrt(deg)
    dinv_ref[...] = dinv
    g_ref[...] = ht * dinv.reshape(1, NP)


def _matmul(wt, x_pad, degp):
    return pl.pallas_call(
        _mm_body,
        out_shape=(
            jax.ShapeDtypeStruct((2, NP), jnp.float32),
            jax.ShapeDtypeStruct((NP,), jnp.float32),
        ),
    )(wt, x_pad, degp)


def _msg_body(row2, col2, ew2, g, dinv, out_hbm,
              rowb, colb, ewb, m0, m1, dinv_v, g0_v, g1_v, ta, tb,
              out0_s, out1_s, sem):
    c = lax.axis_index("c")
    s = lax.axis_index("s")
    w = c * NS + s

    is0 = jnp.where(c == 0, jnp.float32(1.0), jnp.float32(0.0))
    pltpu.sync_copy(dinv.at[pl.ds(s * CHUNK, CHUNK)], ta)
    pltpu.sync_copy(g.at[0, pl.ds(s * CHUNK, CHUNK)], tb)

    def il0(i, _):
        y = ta[pl.ds(i * L, L)]
        tb[pl.ds(i * L, L)] = is0 * y * tb[pl.ds(i * L, L)]
        return 0
    lax.fori_loop(0, CHUNK // L, il0, 0)
    pltpu.sync_copy(tb, out0_s.at[pl.ds(s * CHUNK, CHUNK)])
    pltpu.sync_copy(g.at[1, pl.ds(s * CHUNK, CHUNK)], tb)

    def il1(i, _):
        y = ta[pl.ds(i * L, L)]
        tb[pl.ds(i * L, L)] = is0 * y * tb[pl.ds(i * L, L)]
        return 0
    lax.fori_loop(0, CHUNK // L, il1, 0)
    pltpu.sync_copy(tb, out1_s.at[pl.ds(s * CHUNK, CHUNK)])
    plsc.subcore_barrier()

    pltpu.sync_copy(dinv, dinv_v)
    pltpu.sync_copy(g.at[0], g0_v)
    pltpu.sync_copy(g.at[1], g1_v)
    pltpu.sync_copy(row2.at[pl.ds(w * WROWS, WROWS)], rowb)
    pltpu.sync_copy(col2.at[pl.ds(w * WROWS, WROWS)], colb)
    pltpu.sync_copy(ew2.at[pl.ds(w * WROWS, WROWS)], ewb)

    def ml(i, _):
        for j in range(128 // L):
            r16 = rowb[i, pl.ds(j * L, L)]
            c16 = colb[i, pl.ds(j * L, L)]
            w16 = ewb[i, pl.ds(j * L, L)]
            dc = plsc.load_gather(dinv_v, [c16])
            g0g = plsc.load_gather(g0_v, [r16])
            g1g = plsc.load_gather(g1_v, [r16])
            nv = w16 * dc
            m0[i, pl.ds(j * L, L)] = nv * g0g
            m1[i, pl.ds(j * L, L)] = nv * g1g
        pltpu.async_copy(m0.at[i], out0_s.at[colb.at[i]], sem, add=True)
        pltpu.async_copy(m1.at[i], out1_s.at[colb.at[i]], sem, add=True)
        return 0
    lax.fori_loop(0, WROWS, ml, 0)
    pltpu.make_async_copy(ew2.at[pl.ds(0, WROWS)], m0, sem).wait()
    pltpu.make_async_copy(ew2.at[pl.ds(0, WROWS)], m1, sem).wait()
    plsc.subcore_barrier()

    pltpu.sync_copy(out0_s.at[pl.ds(s * CHUNK, CHUNK)], ta)
    pltpu.sync_copy(ta, out_hbm.at[c, 0, pl.ds(s * CHUNK, CHUNK)])
    pltpu.sync_copy(out1_s.at[pl.ds(s * CHUNK, CHUNK)], ta)
    pltpu.sync_copy(ta, out_hbm.at[c, 1, pl.ds(s * CHUNK, CHUNK)])


def _msg_call(row2, col2, ew2, g, dinv1d):
    return pl.kernel(
        _msg_body,
        out_type=jax.ShapeDtypeStruct((NC, 2, NP), jnp.float32),
        mesh=_mesh(),
        compiler_params=pltpu.CompilerParams(needs_layout_passes=False),
        scratch_types=[
            pltpu.VMEM((WROWS, 128), jnp.int32),
            pltpu.VMEM((WROWS, 128), jnp.int32),
            pltpu.VMEM((WROWS, 128), jnp.float32),
            pltpu.VMEM((WROWS, 128), jnp.float32),
            pltpu.VMEM((WROWS, 128), jnp.float32),
            pltpu.VMEM((NP,), jnp.float32),
            pltpu.VMEM((NP,), jnp.float32),
            pltpu.VMEM((NP,), jnp.float32),
            pltpu.VMEM((CHUNK,), jnp.float32),
            pltpu.VMEM((CHUNK,), jnp.float32),
            pltpu.VMEM_SHARED((NP,), jnp.float32),
            pltpu.VMEM_SHARED((NP,), jnp.float32),
            pltpu.SemaphoreType.DMA,
        ],
    )(row2, col2, ew2, g, dinv1d)


def _topk_body(op_ref, b_ref, attn_ref, fcb_ref, f0_ref, f1_ref, o_ref):
    b0, b1 = b_ref[0], b_ref[1]
    a0, a1 = attn_ref[0], attn_ref[1]
    fcb = fcb_ref[0]
    na = jnp.sqrt(a0 * a0 + a1 * a1)

    o0 = jnp.maximum(op_ref[0, 0] + op_ref[1, 0] + b0, 0.0)
    o1 = jnp.maximum(op_ref[0, 1] + op_ref[1, 1] + b1, 0.0)
    z = (o0 * a0 + o1 * a1) / na
    score = jnp.tanh(z)

    rr = lax.broadcasted_iota(jnp.int32, (OROWS, 128), 0)
    cc = lax.broadcasted_iota(jnp.int32, (OROWS, 128), 1)
    valid = (rr * 128 + cc) < N
    sb = lax.bitcast_convert_type(z, jnp.int32)
    key = jnp.where(sb >= 0, sb, jnp.bitwise_xor(~sb, jnp.int32(INT_MIN)))
    key = jnp.where(valid, key, jnp.int32(INT_MIN))
    p0 = jnp.where(valid, score * o0, 0.0)
    p1 = jnp.where(valid, score * o1, 0.0)

    pad_i = jnp.full((SROWS - OROWS, 128), INT_MIN, jnp.int32)
    pad_f = jnp.zeros((SROWS - OROWS, 128), jnp.float32)
    K = jnp.concatenate([key, pad_i], axis=0)
    P0 = jnp.concatenate([p0, pad_f], axis=0)
    P1 = jnp.concatenate([p1, pad_f], axis=0)
    R = lax.broadcasted_iota(jnp.int32, (SROWS, 128), 0)
    C = lax.broadcasted_iota(jnp.int32, (SROWS, 128), 1)
    I = R * 128 + C

    def xshuf(x, j):
        if j < 128:
            lo = (C & j) == 0
            return jnp.where(lo, pltpu.roll(x, 128 - j, 1),
                             pltpu.roll(x, j, 1))
        m = j // 128
        lo = (R & m) == 0
        return jnp.where(lo, pltpu.roll(x, SROWS - m, 0),
                         pltpu.roll(x, m, 0))

    def bit_set(j):
        return ((C & j) != 0) if j < 128 else ((R & (j // 128)) != 0)

    k = 2
    while k <= NSORT:
        j = k // 2
        while j >= 1:
            Kp, Ip = xshuf(K, j), xshuf(I, j)
            P0p, P1p = xshuf(P0, j), xshuf(P1, j)
            before = (K > Kp) | ((K == Kp) & (I < Ip))
            is_low = ~bit_set(j)
            dir_asc = ~bit_set(k)
            cond = before == (is_low == dir_asc)
            K = jnp.where(cond, K, Kp)
            I = jnp.where(cond, I, Ip)
            P0 = jnp.where(cond, P0, P0p)
            P1 = jnp.where(cond, P1, P1p)
            j //= 2
        k *= 2

    ypre = jnp.sum(P0 * f0_ref[...] + P1 * f1_ref[...]) + fcb
    y = jnp.float32(1.0) / (jnp.float32(1.0) + jnp.exp(-ypre))
    o_ref[...] = jnp.full((8, 128), y, jnp.float32)


def _topk(out_part, b, attn, fc_b, f0, f1):
    return pl.pallas_call(
        _topk_body,
        in_specs=[
            pl.BlockSpec(memory_space=pltpu.MemorySpace.VMEM),
            pl.BlockSpec(memory_space=pltpu.SMEM),
            pl.BlockSpec(memory_space=pltpu.SMEM),
            pl.BlockSpec(memory_space=pltpu.SMEM),
            pl.BlockSpec(memory_space=pltpu.MemorySpace.VMEM),
            pl.BlockSpec(memory_space=pltpu.MemorySpace.VMEM),
        ],
        out_shape=jax.ShapeDtypeStruct((8, 128), jnp.float32),
    )(out_part, b, attn, fc_b, f0, f1)


def kernel(x, edge_list, edge_attr, W, b, attn, fc_w, fc_b):
    row = edge_list[0].astype(jnp.int32)
    col = edge_list[1].astype(jnp.int32)
    ew = edge_attr.astype(jnp.float32)

    npad = EPAD - E
    rowp = jnp.concatenate([row, jnp.zeros((npad,), jnp.int32)])
    colp = jnp.concatenate(
        [col, N + (jnp.arange(npad, dtype=jnp.int32) % (NP - N))])
    ewp = jnp.concatenate([ew, jnp.zeros((npad,), jnp.float32)])
    row2 = rowp.reshape(EROWS, 128)
    col2 = colp.reshape(EROWS, 128)
    ew2 = ewp.reshape(EROWS, 128)

    x_pad = jnp.pad(x, ((0, NP - N), (0, 0)))
    wt = W.T

    degp = _deg_call(col2, ew2)
    g, dinv1d = _matmul(wt, x_pad, degp)
    out_part = _msg_call(row2, col2, ew2, g, dinv1d)

    fr = fc_w.reshape(N, 2)
    f0 = jnp.pad(fr[:, 0], (0, NSORT - N)).reshape(SROWS, 128)
    f1 = jnp.pad(fr[:, 1], (0, NSORT - N)).reshape(SROWS, 128)

    yblk = _topk(out_part.reshape(NC, 2, OROWS, 128), b, attn, fc_b, f0, f1)
    return yblk[0, 0].reshape(1)
